# Initial kernel scaffold; baseline (speedup 1.0000x reference)
import jax
import jax.numpy as jnp
from jax.experimental import pallas as pl


def kernel(x, edge_index, W1, W2, Wr, br):
    raise NotImplementedError("write your pallas kernel here")


# speedup vs baseline: 4.8168x; 4.8168x over previous
import functools

import jax
import jax.numpy as jnp
from jax import lax
from jax.experimental import pallas as pl
from jax.experimental.pallas import tpu as pltpu
from jax.experimental.pallas import tpu_sc as plsc

N = 10000
E = 320000
D = 128
H = 128

NC = 2
NS = 16
EPT = E // NS
CH = 80
NCHUNK = EPT // CH
NPAD = 10240
RPT = NPAD // NS
ZR = 128
CPT = 640

_MM_BLK = 1000


def _mm3_body(x_ref, w1_ref, w2_ref, wr_ref, br_ref, h1_ref, h2_ref, xr_ref):
    xb = x_ref[...]
    dn = (((1,), (1,)), ((), ()))
    h1_ref[...] = lax.dot_general(xb, w1_ref[...], dn,
                                  preferred_element_type=jnp.float32)
    h2_ref[...] = lax.dot_general(xb, w2_ref[...], dn,
                                  preferred_element_type=jnp.float32)
    xr_ref[...] = lax.dot_general(xb, wr_ref[...], dn,
                                  preferred_element_type=jnp.float32) + br_ref[...]


def _combine_body(xr_ref, s1_ref, c1_ref, s2_ref, c2_ref, o_ref):
    c1 = jnp.maximum(c1_ref[...], 1.0)
    c2 = jnp.maximum(c2_ref[...], 1.0)
    o_ref[...] = jnp.maximum(
        xr_ref[...] + s1_ref[...] / c1 + s2_ref[...] / c2, 0.0)


_sc_mesh = plsc.VectorSubcoreMesh(
    core_axis_name="c", subcore_axis_name="s", num_cores=NC, num_subcores=NS)


@functools.partial(
    pl.kernel,
    out_type=(
        jax.ShapeDtypeStruct((NPAD, D), jnp.float32),
        jax.ShapeDtypeStruct((NPAD,), jnp.float32),
        jax.ShapeDtypeStruct((NPAD, D), jnp.float32),
        jax.ShapeDtypeStruct((NPAD,), jnp.float32),
    ),
    mesh=_sc_mesh,
    scratch_types=[
        pltpu.VMEM((CH,), jnp.int32),
        pltpu.VMEM((CH,), jnp.int32),
        pltpu.VMEM((CH, D), jnp.float32),
        pltpu.VMEM((CH,), jnp.float32),
        pltpu.VMEM((ZR, D), jnp.float32),
        pltpu.VMEM((CPT,), jnp.float32),
        pltpu.VMEM_SHARED((NPAD, D), jnp.float32),
        pltpu.VMEM_SHARED((NPAD,), jnp.float32),
        pltpu.SemaphoreType.DMA,
    ],
)
def _sc_segsum(h1, h2, src, dst, sum1, cnt1, sum2, cnt2,
               gidx, sidx, rows, ones_v, zrow, zcnt, acc_sh, cnt_sh, sem):
    s = lax.axis_index("s")
    c = lax.axis_index("c")

    zv = jnp.zeros((16,), jnp.float32)
    ov = jnp.ones((16,), jnp.float32)

    def fill_zrow(k, _):
        zrow[k // 8, pl.ds((k % 8) * 16, 16)] = zv
        return 0
    lax.fori_loop(0, ZR * (D // 16), fill_zrow, 0)

    def fill_zcnt(k, _):
        zcnt[pl.ds(k * 16, 16)] = zv
        return 0
    lax.fori_loop(0, CPT // 16, fill_zcnt, 0)

    def fill_ones(k, _):
        ones_v[pl.ds(k * 16, 16)] = ov
        return 0
    lax.fori_loop(0, CH // 16, fill_ones, 0)

    for t in range(RPT // ZR):
        pltpu.sync_copy(zrow, acc_sh.at[pl.ds(s * RPT + t * ZR, ZR)])
    pltpu.sync_copy(zcnt, cnt_sh.at[pl.ds(s * CPT, CPT)])
    plsc.subcore_barrier()

    def run_direction(h_hbm, g_hbm, t_hbm, sum_out, cnt_out):
        def body(i, _):
            off = s * EPT + i * CH
            pltpu.sync_copy(g_hbm.at[pl.ds(off, CH)], gidx)
            pltpu.sync_copy(t_hbm.at[pl.ds(off, CH)], sidx)
            pltpu.async_copy(h_hbm.at[gidx], rows, sem).wait()
            pltpu.sync_copy(rows, acc_sh.at[sidx], add=True)
            pltpu.sync_copy(ones_v, cnt_sh.at[sidx], add=True)
            return 0
        lax.fori_loop(0, NCHUNK, body, 0)
        plsc.subcore_barrier()
        for t in range(RPT // ZR):
            r0 = s * RPT + t * ZR
            pltpu.sync_copy(acc_sh.at[pl.ds(r0, ZR)], sum_out.at[pl.ds(r0, ZR)])
        pltpu.sync_copy(cnt_sh.at[pl.ds(s * CPT, CPT)],
                        cnt_out.at[pl.ds(s * CPT, CPT)])

    @pl.when(c == 0)
    def _():
        run_direction(h1, src, dst, sum1, cnt1)

    @pl.when(c == 1)
    def _():
        run_direction(h2, dst, src, sum2, cnt2)


def kernel(x, edge_index, W1, W2, Wr, br):
    src = edge_index[0]
    dst = edge_index[1]
    br2 = br.reshape(1, H)

    nblk = N // _MM_BLK
    h1, h2, xr = pl.pallas_call(
        _mm3_body,
        grid=(nblk,),
        in_specs=[
            pl.BlockSpec((_MM_BLK, D), lambda i: (i, 0)),
            pl.BlockSpec((H, D), lambda i: (0, 0)),
            pl.BlockSpec((H, D), lambda i: (0, 0)),
            pl.BlockSpec((H, D), lambda i: (0, 0)),
            pl.BlockSpec((1, H), lambda i: (0, 0)),
        ],
        out_specs=[
            pl.BlockSpec((_MM_BLK, H), lambda i: (i, 0)),
            pl.BlockSpec((_MM_BLK, H), lambda i: (i, 0)),
            pl.BlockSpec((_MM_BLK, H), lambda i: (i, 0)),
        ],
        out_shape=[
            jax.ShapeDtypeStruct((N, H), jnp.float32),
            jax.ShapeDtypeStruct((N, H), jnp.float32),
            jax.ShapeDtypeStruct((N, H), jnp.float32),
        ],
    )(x, W1, W2, Wr, br2)

    sum1, cnt1, sum2, cnt2 = _sc_segsum(h1, h2, src, dst)

    c1 = cnt1[:N].reshape(N, 1)
    c2 = cnt2[:N].reshape(N, 1)
    out = pl.pallas_call(
        _combine_body,
        grid=(nblk,),
        in_specs=[
            pl.BlockSpec((_MM_BLK, H), lambda i: (i, 0)),
            pl.BlockSpec((_MM_BLK, H), lambda i: (i, 0)),
            pl.BlockSpec((_MM_BLK, 1), lambda i: (i, 0)),
            pl.BlockSpec((_MM_BLK, H), lambda i: (i, 0)),
            pl.BlockSpec((_MM_BLK, 1), lambda i: (i, 0)),
        ],
        out_specs=pl.BlockSpec((_MM_BLK, H), lambda i: (i, 0)),
        out_shape=jax.ShapeDtypeStruct((N, H), jnp.float32),
    )(xr, sum1, c1, sum2, c2)
    return out


# speedup vs baseline: 5.0202x; 1.0422x over previous
import functools

import jax
import jax.numpy as jnp
from jax import lax
from jax.experimental import pallas as pl
from jax.experimental.pallas import tpu as pltpu
from jax.experimental.pallas import tpu_sc as plsc

N = 10000
E = 320000
D = 128
H = 128

NC = 2
NS = 16
CH = 128
TCH = 160
GB = 16
NG = TCH // GB
NCHUNKS = NS * TCH
EP = NCHUNKS * CH
NPAD = 10240
RPT = NPAD // NS
ZR = 128
CPT = NPAD // NS

_MM_BLK = 1000


def _mm3_body(x_ref, w1_ref, w2_ref, wr_ref, br_ref, h1_ref, h2_ref, xr_ref):
    xb = x_ref[...]
    dn = (((1,), (1,)), ((), ()))
    h1_ref[...] = lax.dot_general(xb, w1_ref[...], dn,
                                  preferred_element_type=jnp.float32)
    h2_ref[...] = lax.dot_general(xb, w2_ref[...], dn,
                                  preferred_element_type=jnp.float32)
    xr_ref[...] = lax.dot_general(xb, wr_ref[...], dn,
                                  preferred_element_type=jnp.float32) + br_ref[...]


def _combine_body(xr_ref, s1_ref, c1_ref, s2_ref, c2_ref, o_ref):
    c1 = jnp.maximum(c1_ref[...], 1.0)
    c2 = jnp.maximum(c2_ref[...], 1.0)
    o_ref[...] = jnp.maximum(
        xr_ref[...] + s1_ref[...] / c1 + s2_ref[...] / c2, 0.0)


_sc_mesh = plsc.VectorSubcoreMesh(
    core_axis_name="c", subcore_axis_name="s", num_cores=NC, num_subcores=NS)


@functools.partial(
    pl.kernel,
    out_type=(
        jax.ShapeDtypeStruct((NPAD, D), jnp.float32),
        jax.ShapeDtypeStruct((NPAD,), jnp.float32),
        jax.ShapeDtypeStruct((NPAD, D), jnp.float32),
        jax.ShapeDtypeStruct((NPAD,), jnp.float32),
    ),
    mesh=_sc_mesh,
    scratch_types=[
        pltpu.VMEM((GB, CH), jnp.int32),
        pltpu.VMEM((GB, CH), jnp.int32),
        pltpu.VMEM((GB, CH), jnp.int32),
        pltpu.VMEM((GB, CH), jnp.int32),
        pltpu.VMEM((CH, D), jnp.float32),
        pltpu.VMEM((CH, D), jnp.float32),
        pltpu.VMEM((CH,), jnp.float32),
        pltpu.VMEM((CPT,), jnp.float32),
        pltpu.VMEM_SHARED((NPAD, D), jnp.float32),
        pltpu.VMEM_SHARED((NPAD,), jnp.float32),
        pltpu.SemaphoreType.DMA,
        pltpu.SemaphoreType.DMA,
        pltpu.SemaphoreType.DMA,
        pltpu.SemaphoreType.DMA,
    ],
)
def _sc_segsum(h1, h2, src_g, dst_s, dst_g, src_s, sum1, cnt1, sum2, cnt2,
               gidxA, sidxA, gidxB, sidxB, rows0, rows1, ones_v, zcnt,
               acc_sh, cnt_sh, sem0, sem1, semiA, semiB):
    s = lax.axis_index("s")
    c = lax.axis_index("c")

    zv = jnp.zeros((16,), jnp.float32)
    ov = jnp.ones((16,), jnp.float32)

    def fill_zrow(k, _):
        rows0[k // 8, pl.ds((k % 8) * 16, 16)] = zv
        return 0
    lax.fori_loop(0, ZR * (D // 16), fill_zrow, 0)

    def fill_zcnt(k, _):
        zcnt[pl.ds(k * 16, 16)] = zv
        return 0
    lax.fori_loop(0, CPT // 16, fill_zcnt, 0)

    def fill_ones(k, _):
        ones_v[pl.ds(k * 16, 16)] = ov
        return 0
    lax.fori_loop(0, CH // 16, fill_ones, 0)

    for t in range(RPT // ZR):
        pltpu.sync_copy(rows0, acc_sh.at[pl.ds(s * RPT + t * ZR, ZR)])
    pltpu.sync_copy(zcnt, cnt_sh.at[pl.ds(s * CPT, CPT)])
    plsc.subcore_barrier()

    def run_direction(h_hbm, g2d, s2d, sum_out, cnt_out):
        idx_bufs = ((gidxA, sidxA, semiA), (gidxB, sidxB, semiB))
        bufs = ((rows0, sem0), (rows1, sem1))

        def stage(gq, p):
            gI, sI, smi = idx_bufs[p]
            r0 = s * TCH + gq * GB
            pltpu.async_copy(g2d.at[pl.ds(r0, GB)], gI, smi)
            pltpu.async_copy(s2d.at[pl.ds(r0, GB)], sI, smi)

        def wait_stage(gq, p):
            gI, sI, smi = idx_bufs[p]
            r0 = s * TCH + gq * GB
            pltpu.make_async_copy(g2d.at[pl.ds(r0, GB)], gI, smi).wait()
            pltpu.make_async_copy(s2d.at[pl.ds(r0, GB)], sI, smi).wait()

        stage(0, 0)

        def outer(t, _):
            for p in range(2):
                g = 2 * t + p
                wait_stage(g, p)

                @pl.when(g + 1 < NG)
                def _():
                    stage(g + 1, 1 - p)

                gI, sI, _ = idx_bufs[p]
                pltpu.async_copy(h_hbm.at[gI.at[0]], rows0, sem0)

                def pair(u, _):
                    for b in range(2):
                        i = 2 * u + b
                        rb, sg = bufs[b]
                        ro, so = bufs[1 - b]
                        pltpu.make_async_copy(h_hbm.at[gI.at[i]], rb, sg).wait()

                        @pl.when(i + 1 < GB)
                        def _():
                            pltpu.async_copy(h_hbm.at[gI.at[i + 1]], ro, so)

                        pltpu.sync_copy(rb, acc_sh.at[sI.at[i]], add=True)
                        pltpu.sync_copy(ones_v, cnt_sh.at[sI.at[i]], add=True)
                    return 0
                lax.fori_loop(0, GB // 2, pair, 0)
            return 0
        lax.fori_loop(0, NG // 2, outer, 0)

        plsc.subcore_barrier()
        for t in range(RPT // ZR):
            r0 = s * RPT + t * ZR
            pltpu.sync_copy(acc_sh.at[pl.ds(r0, ZR)], sum_out.at[pl.ds(r0, ZR)])
        pltpu.sync_copy(cnt_sh.at[pl.ds(s * CPT, CPT)],
                        cnt_out.at[pl.ds(s * CPT, CPT)])

    @pl.when(c == 0)
    def _():
        run_direction(h1, src_g, dst_s, sum1, cnt1)

    @pl.when(c == 1)
    def _():
        run_direction(h2, dst_g, src_s, sum2, cnt2)


def kernel(x, edge_index, W1, W2, Wr, br):
    src = edge_index[0]
    dst = edge_index[1]
    pad_g = jnp.zeros((EP - E,), jnp.int32)
    pad_s = jnp.full((EP - E,), NPAD - 1, jnp.int32)
    src_g = jnp.concatenate([src, pad_g]).reshape(NCHUNKS, CH)
    dst_s = jnp.concatenate([dst, pad_s]).reshape(NCHUNKS, CH)
    dst_g = jnp.concatenate([dst, pad_g]).reshape(NCHUNKS, CH)
    src_s = jnp.concatenate([src, pad_s]).reshape(NCHUNKS, CH)
    br2 = br.reshape(1, H)

    nblk = N // _MM_BLK
    h1, h2, xr = pl.pallas_call(
        _mm3_body,
        grid=(nblk,),
        in_specs=[
            pl.BlockSpec((_MM_BLK, D), lambda i: (i, 0)),
            pl.BlockSpec((H, D), lambda i: (0, 0)),
            pl.BlockSpec((H, D), lambda i: (0, 0)),
            pl.BlockSpec((H, D), lambda i: (0, 0)),
            pl.BlockSpec((1, H), lambda i: (0, 0)),
        ],
        out_specs=[
            pl.BlockSpec((_MM_BLK, H), lambda i: (i, 0)),
            pl.BlockSpec((_MM_BLK, H), lambda i: (i, 0)),
            pl.BlockSpec((_MM_BLK, H), lambda i: (i, 0)),
        ],
        out_shape=[
            jax.ShapeDtypeStruct((N, H), jnp.float32),
            jax.ShapeDtypeStruct((N, H), jnp.float32),
            jax.ShapeDtypeStruct((N, H), jnp.float32),
        ],
    )(x, W1, W2, Wr, br2)

    sum1, cnt1, sum2, cnt2 = _sc_segsum(h1, h2, src_g, dst_s, dst_g, src_s)

    c1 = cnt1[:N].reshape(N, 1)
    c2 = cnt2[:N].reshape(N, 1)
    out = pl.pallas_call(
        _combine_body,
        grid=(nblk,),
        in_specs=[
            pl.BlockSpec((_MM_BLK, H), lambda i: (i, 0)),
            pl.BlockSpec((_MM_BLK, H), lambda i: (i, 0)),
            pl.BlockSpec((_MM_BLK, 1), lambda i: (i, 0)),
            pl.BlockSpec((_MM_BLK, H), lambda i: (i, 0)),
            pl.BlockSpec((_MM_BLK, 1), lambda i: (i, 0)),
        ],
        out_specs=pl.BlockSpec((_MM_BLK, H), lambda i: (i, 0)),
        out_shape=jax.ShapeDtypeStruct((N, H), jnp.float32),
    )(xr, sum1, c1, sum2, c2)
    return out
